# baseline (speedup 1.0000x reference)
import jax
import jax.numpy as jnp
from jax.experimental import pallas as pl
from jax.experimental.pallas import tpu as pltpu

B = 8
C_IN = 384
HW = 576
D = 64
K = 1024
BETA = 0.25
G = 4


def _vq_body(z_ref, win_ref, bin_ref, cb_ref, wout_ref, bout_ref,
             out_ref, loss_ref, cba_ref):
    b = pl.program_id(0)
    cb = cb_ref[...]

    @pl.when(b == 0)
    def _prep():
        cba_ref[:, 0:D] = -2.0 * cb
        cba_ref[:, D:D + 1] = jnp.sum(cb * cb, axis=1, keepdims=True)

    part = jnp.zeros((1, 1), jnp.float32)
    for g in range(G):
        zb = z_ref[g]
        zp_aug = jnp.dot(win_ref[...], zb,
                         preferred_element_type=jnp.float32)
        zp_aug = zp_aug + bin_ref[...]
        dt = jnp.dot(cba_ref[...], zp_aug,
                     preferred_element_type=jnp.float32)
        iota = jax.lax.broadcasted_iota(jnp.int32, (K, HW), 0)
        keyi = (jax.lax.bitcast_convert_type(dt, jnp.int32)
                & jnp.int32(-1024))
        keyf = jax.lax.bitcast_convert_type(keyi | iota, jnp.float32)
        kmin = jnp.min(keyf, axis=0, keepdims=True)
        onehot = (keyf == kmin).astype(jnp.float32)
        zq = jax.lax.dot_general(cb, onehot, (((0,), (0,)), ((), ())),
                                 preferred_element_type=jnp.float32)
        minv = jax.lax.bitcast_convert_type(
            jax.lax.bitcast_convert_type(kmin, jnp.int32) & jnp.int32(-1024),
            jnp.float32)
        zp = zp_aug[0:D]
        ze2 = jnp.sum(zp * zp, axis=0, keepdims=True)
        part = part + jnp.sum(minv + ze2) * ((1.0 + BETA) / (B * HW * D))
        out = jnp.dot(wout_ref[...], zq,
                      preferred_element_type=jnp.float32)
        out_ref[g] = out + bout_ref[...]

    @pl.when(b == 0)
    def _init():
        loss_ref[...] = jnp.zeros_like(loss_ref)

    loss_ref[...] += part


def kernel(z, W_in, b_in, codebook, W_out, b_out):
    z3 = z.reshape(B, C_IN, HW)
    win_aug = jnp.concatenate(
        [W_in, jnp.zeros((1, C_IN), jnp.float32)], axis=0)
    bin_aug = jnp.concatenate(
        [b_in, jnp.ones((1,), jnp.float32)], axis=0).reshape(D + 1, 1)
    out3, loss = pl.pallas_call(
        _vq_body,
        grid=(B // G,),
        in_specs=[
            pl.BlockSpec((G, C_IN, HW), lambda b: (b, 0, 0)),
            pl.BlockSpec((D + 1, C_IN), lambda b: (0, 0)),
            pl.BlockSpec((D + 1, 1), lambda b: (0, 0)),
            pl.BlockSpec((K, D), lambda b: (0, 0)),
            pl.BlockSpec((C_IN, D), lambda b: (0, 0)),
            pl.BlockSpec((C_IN, 1), lambda b: (0, 0)),
        ],
        out_specs=[
            pl.BlockSpec((G, C_IN, HW), lambda b: (b, 0, 0)),
            pl.BlockSpec((1, 1), lambda b: (0, 0)),
        ],
        out_shape=[
            jax.ShapeDtypeStruct((B, C_IN, HW), jnp.float32),
            jax.ShapeDtypeStruct((1, 1), jnp.float32),
        ],
        scratch_shapes=[pltpu.VMEM((K, D + 1), jnp.float32)],
    )(z3, win_aug, bin_aug, codebook, W_out, b_out.reshape(C_IN, 1))
    return loss[0, 0], out3.reshape(B, C_IN, 24, 24)


# speedup vs baseline: 1.4935x; 1.4935x over previous
import jax
import jax.numpy as jnp
from jax.experimental import pallas as pl
from jax.experimental.pallas import tpu as pltpu

B = 8
C_IN = 384
H = 24
W = 24
N = B * H * W
D = 64
K = 1024
BETA = 0.25
R = 576


def _vq_body(z_ref, win_ref, bin_ref, cb_ref, wout_ref, bout_ref,
             out_ref, loss_ref, cba_ref):
    step = pl.program_id(0)
    cb = cb_ref[...]

    @pl.when(step == 0)
    def _prep():
        cba_ref[:, 0:D] = -2.0 * cb
        cba_ref[:, D:D + 1] = jnp.sum(cb * cb, axis=1, keepdims=True)

    zr = z_ref[...]
    ze_aug = jax.lax.dot_general(
        zr, win_ref[...], (((1,), (1,)), ((), ())),
        preferred_element_type=jnp.float32) + bin_ref[...]
    dt = jax.lax.dot_general(
        ze_aug, cba_ref[...], (((1,), (1,)), ((), ())),
        preferred_element_type=jnp.float32)
    iota = jax.lax.broadcasted_iota(jnp.int32, (R, K), 1)
    keyi = jax.lax.bitcast_convert_type(dt, jnp.int32) & jnp.int32(-1024)
    keyf = jax.lax.bitcast_convert_type(keyi | iota, jnp.float32)
    kmin = jnp.min(keyf, axis=1, keepdims=True)
    onehot = (keyf == kmin).astype(jnp.float32)
    zq = jax.lax.dot_general(onehot, cb, (((1,), (0,)), ((), ())),
                             preferred_element_type=jnp.float32)
    minv = jax.lax.bitcast_convert_type(
        jax.lax.bitcast_convert_type(kmin, jnp.int32) & jnp.int32(-1024),
        jnp.float32)
    ze = ze_aug[:, 0:D]
    ze2 = jnp.sum(ze * ze, axis=1, keepdims=True)
    part = jnp.sum(minv + ze2) * ((1.0 + BETA) / (N * D))

    @pl.when(step == 0)
    def _init():
        loss_ref[...] = jnp.zeros_like(loss_ref)

    loss_ref[...] += part
    out = jax.lax.dot_general(zq, wout_ref[...], (((1,), (1,)), ((), ())),
                              preferred_element_type=jnp.float32)
    out_ref[...] = out + bout_ref[...]


def kernel(z, W_in, b_in, codebook, W_out, b_out):
    z_rows = z.transpose(0, 2, 3, 1).reshape(N, C_IN)
    win_aug = jnp.concatenate(
        [W_in, jnp.zeros((1, C_IN), jnp.float32)], axis=0)
    bin_aug = jnp.concatenate(
        [b_in, jnp.ones((1,), jnp.float32)], axis=0).reshape(1, D + 1)
    out_rows, loss = pl.pallas_call(
        _vq_body,
        grid=(N // R,),
        in_specs=[
            pl.BlockSpec((R, C_IN), lambda i: (i, 0)),
            pl.BlockSpec((D + 1, C_IN), lambda i: (0, 0)),
            pl.BlockSpec((1, D + 1), lambda i: (0, 0)),
            pl.BlockSpec((K, D), lambda i: (0, 0)),
            pl.BlockSpec((C_IN, D), lambda i: (0, 0)),
            pl.BlockSpec((1, C_IN), lambda i: (0, 0)),
        ],
        out_specs=[
            pl.BlockSpec((R, C_IN), lambda i: (i, 0)),
            pl.BlockSpec((1, 1), lambda i: (0, 0)),
        ],
        out_shape=[
            jax.ShapeDtypeStruct((N, C_IN), jnp.float32),
            jax.ShapeDtypeStruct((1, 1), jnp.float32),
        ],
        scratch_shapes=[pltpu.VMEM((K, D + 1), jnp.float32)],
    )(z_rows, win_aug, bin_aug, codebook, W_out, b_out.reshape(1, C_IN))
    out = out_rows.reshape(B, H, W, C_IN).transpose(0, 3, 1, 2)
    return loss[0, 0], out


# speedup vs baseline: 1.6002x; 1.0714x over previous
import jax
import jax.numpy as jnp
from jax.experimental import pallas as pl
from jax.experimental.pallas import tpu as pltpu

B = 8
C_IN = 384
H = 24
W = 24
N = B * H * W
D = 64
K = 1024
BETA = 0.25
R = 1152


def _vq_body(z_ref, win_ref, bin_ref, cb_ref, wout_ref, bout_ref,
             out_ref, loss_ref, cba_ref, iota_ref):
    step = pl.program_id(0)
    cb = cb_ref[...]

    @pl.when(step == 0)
    def _prep():
        cba_ref[:, 0:D] = -2.0 * cb
        cba_ref[:, D:D + 1] = jnp.sum(cb * cb, axis=1, keepdims=True)
        iota_ref[...] = jax.lax.broadcasted_iota(jnp.int32, (8, K), 1)

    zr = z_ref[...]
    ze_aug = jax.lax.dot_general(
        zr, win_ref[...], (((1,), (1,)), ((), ())),
        preferred_element_type=jnp.float32) + bin_ref[...]
    dt = jax.lax.dot_general(
        ze_aug, cba_ref[...], (((1,), (1,)), ((), ())),
        preferred_element_type=jnp.float32)
    iota = jnp.broadcast_to(iota_ref[0:1], (R, K))
    keyi = jax.lax.bitcast_convert_type(dt, jnp.int32) & jnp.int32(-1024)
    keyf = jax.lax.bitcast_convert_type(keyi | iota, jnp.float32)
    m = keyf[:, 0:128]
    for c in range(1, 8):
        m = jnp.minimum(m, keyf[:, 128 * c:128 * (c + 1)])
    kmin = jnp.min(m, axis=1, keepdims=True)
    onehot = (keyf == kmin).astype(jnp.float32)
    zq = jax.lax.dot_general(onehot, cb, (((1,), (0,)), ((), ())),
                             preferred_element_type=jnp.float32)
    minv = jax.lax.bitcast_convert_type(
        jax.lax.bitcast_convert_type(kmin, jnp.int32) & jnp.int32(-1024),
        jnp.float32)
    ze = ze_aug[:, 0:D]
    ze2 = jnp.sum(ze * ze, axis=1, keepdims=True)
    part = jnp.sum(minv + ze2) * ((1.0 + BETA) / (N * D))

    @pl.when(step == 0)
    def _init():
        loss_ref[...] = jnp.zeros_like(loss_ref)

    loss_ref[...] += part
    out = jax.lax.dot_general(zq, wout_ref[...], (((1,), (1,)), ((), ())),
                              preferred_element_type=jnp.float32)
    out_ref[...] = out + bout_ref[...]


def kernel(z, W_in, b_in, codebook, W_out, b_out):
    z_rows = z.transpose(0, 2, 3, 1).reshape(N, C_IN)
    win_aug = jnp.concatenate(
        [W_in, jnp.zeros((1, C_IN), jnp.float32)], axis=0)
    bin_aug = jnp.concatenate(
        [b_in, jnp.ones((1,), jnp.float32)], axis=0).reshape(1, D + 1)
    out_rows, loss = pl.pallas_call(
        _vq_body,
        grid=(N // R,),
        in_specs=[
            pl.BlockSpec((R, C_IN), lambda i: (i, 0)),
            pl.BlockSpec((D + 1, C_IN), lambda i: (0, 0)),
            pl.BlockSpec((1, D + 1), lambda i: (0, 0)),
            pl.BlockSpec((K, D), lambda i: (0, 0)),
            pl.BlockSpec((C_IN, D), lambda i: (0, 0)),
            pl.BlockSpec((1, C_IN), lambda i: (0, 0)),
        ],
        out_specs=[
            pl.BlockSpec((R, C_IN), lambda i: (i, 0)),
            pl.BlockSpec((1, 1), lambda i: (0, 0)),
        ],
        out_shape=[
            jax.ShapeDtypeStruct((N, C_IN), jnp.float32),
            jax.ShapeDtypeStruct((1, 1), jnp.float32),
        ],
        scratch_shapes=[pltpu.VMEM((K, D + 1), jnp.float32),
                        pltpu.VMEM((8, K), jnp.int32)],
    )(z_rows, win_aug, bin_aug, codebook, W_out, b_out.reshape(1, C_IN))
    out = out_rows.reshape(B, H, W, C_IN).transpose(0, 3, 1, 2)
    return loss[0, 0], out
